# baseline (speedup 1.0000x reference)
import functools

import jax
import jax.numpy as jnp
from jax import lax
from jax.experimental import pallas as pl
from jax.experimental.pallas import tpu as pltpu
from jax.experimental.pallas import tpu_sc as plsc

_N = 10000
_NP = 10240
_D = 128
_E = 320000
_G = 64
_OUT = 64
_ECH = 512
_NCH = _E // _ECH
_CPT = 4
_W = _CPT * _NP
_ERS = 2 * _ECH


def _sc_agg_body(h_hbm, e_hbm, out_hbm, ering, hcol, acc, e0, e1, e2, e3):
    c = lax.axis_index("c")
    s = lax.axis_index("s")
    t = s * 2 + c
    base = t * _W
    esems = (e0, e1, e2, e3)
    zero16 = jnp.zeros((16,), jnp.float32)
    iota16 = lax.iota(jnp.int32, 16)

    pltpu.sync_copy(h_hbm.at[pl.ds(base, _W)], hcol)

    @plsc.parallel_loop(0, _W // 16)
    def _z(i):
        acc[pl.ds(i * 16, 16)] = zero16

    def refill(j, slot):
        pltpu.async_copy(e_hbm.at[jnp.minimum(j, _NCH - 1), 0],
                         ering.at[pl.ds(_ERS * slot, _ERS)], esems[slot])

    def wait_refill(slot):
        pltpu.make_async_copy(e_hbm.at[0, 0],
                              ering.at[pl.ds(_ERS * slot, _ERS)],
                              esems[slot]).wait()

    def do_chunk(slot):
        sb = _ERS * slot

        @plsc.parallel_loop(0, _ECH // 16, unroll=4)
        def grp(g):
            gb = g * 16
            pair = plsc.load_gather(ering, [iota16 + (sb + gb)])
            src16 = jnp.bitwise_and(pair, 0xFFFF)
            dst16 = lax.shift_right_logical(pair, 16)
            w16 = plsc.bitcast(
                plsc.load_gather(ering, [iota16 + (sb + _ECH + gb)]),
                jnp.float32)
            for k in range(_CPT):
                off = jnp.full((16,), k * _NP, jnp.int32)
                v = plsc.load_gather(hcol, [src16 + off]) * w16
                plsc.addupdate_scatter(acc, [dst16 + off], v)

    for slot in range(4):
        refill(slot, slot)

    def rnd(r, carry):
        for slot in range(4):
            wait_refill(slot)
            do_chunk(slot)
            refill(4 * r + slot + 4, slot)
        return carry

    lax.fori_loop(0, _NCH // 4, rnd, 0)
    wait_refill(0)
    do_chunk(0)
    for slot in range(1, 4):
        wait_refill(slot)

    pltpu.sync_copy(acc, out_hbm.at[pl.ds(base, _W)])


def _sc_agg(hT_flat, edata):
    return pl.kernel(
        _sc_agg_body,
        out_type=jax.ShapeDtypeStruct((_D * _NP,), jnp.float32),
        mesh=plsc.VectorSubcoreMesh(core_axis_name="c", subcore_axis_name="s"),
        scratch_types=[
            pltpu.VMEM((4 * _ERS,), jnp.int32),
            pltpu.VMEM((_W,), jnp.float32),
            pltpu.VMEM((_W,), jnp.float32),
            pltpu.SemaphoreType.DMA,
            pltpu.SemaphoreType.DMA,
            pltpu.SemaphoreType.DMA,
            pltpu.SemaphoreType.DMA,
        ],
        compiler_params=pltpu.CompilerParams(needs_layout_passes=False),
    )(hT_flat, edata)


_TDN = (((0,), (0,)), ((), ()))


def _tc_transpose_body(x_ref, o_ref):
    o_ref[...] = x_ref[...].T


def _tc_transpose(x):
    return pl.pallas_call(
        _tc_transpose_body,
        out_shape=jax.ShapeDtypeStruct((_D, _NP), jnp.float32),
    )(x)


def _tc_layer_body(p_ref, h_ref, wr_ref, wt_ref, b_ref, o_ref):
    out = lax.dot_general(wr_ref[...], p_ref[...], _TDN,
                          preferred_element_type=jnp.float32)
    out = out + lax.dot_general(wt_ref[...], h_ref[...], _TDN,
                                preferred_element_type=jnp.float32)
    out = jnp.maximum(out + b_ref[...], 0.0)
    o_ref[...] = out


def _tc_layer(p, h, Wr, Wt, b):
    return pl.pallas_call(
        _tc_layer_body,
        out_shape=jax.ShapeDtypeStruct((_D, _NP), jnp.float32),
    )(p, h, Wr, Wt, b)


def _layer_pool_head_body(p_ref, h_ref, wr_ref, wt_ref, b_ref, b2d_ref,
                          w1_ref, b1_ref, w2_ref, b2_ref, o_ref):
    hblk = lax.dot_general(wr_ref[...], p_ref[...], _TDN,
                           preferred_element_type=jnp.float32)
    hblk = hblk + lax.dot_general(wt_ref[...], h_ref[...], _TDN,
                                  preferred_element_type=jnp.float32)
    hblk = hblk + b_ref[...]

    bvec = b2d_ref[...]
    oh = (lax.broadcasted_iota(jnp.int32, (_G, _NP), 0) == bvec
          ).astype(jnp.float32)
    sums = lax.dot_general(oh, hblk, (((1,), (1,)), ((), ())),
                           preferred_element_type=jnp.float32)
    cnts = jnp.sum(oh, axis=1, keepdims=True)
    pooled = sums / jnp.maximum(cnts, 1.0)
    r = jnp.dot(pooled, w1_ref[...], preferred_element_type=jnp.float32)
    r = jnp.maximum(r + b1_ref[...], 0.0)
    o_ref[...] = (jnp.dot(r, w2_ref[...], preferred_element_type=jnp.float32)
                  + b2_ref[...])


def _layer_pool_head(p, h, Wr, Wt, b, batch2d, w1, b1, w2, b2):
    return pl.pallas_call(
        _layer_pool_head_body,
        out_shape=jax.ShapeDtypeStruct((_G, _OUT), jnp.float32),
    )(p, h, Wr, Wt, b, batch2d, w1, b1, w2, b2)


def kernel(x, edge_index, edge_weight, batch, W_rel0, b_rel0, W_root0,
           W_rel1, b_rel1, W_root1, W_rel2, b_rel2, W_root2,
           lin1_W, lin1_b, lin2_W, lin2_b):
    wsi = lax.bitcast_convert_type(edge_weight, jnp.int32)
    pair = jnp.bitwise_or(edge_index[0],
                          jnp.left_shift(edge_index[1], 16))
    edata = jnp.concatenate(
        [pair.reshape(_NCH, 1, _ECH),
         wsi.reshape(_NCH, 1, _ECH)], axis=1).reshape(_NCH, 1, _ERS)
    batch2d = jnp.pad(batch, (0, _NP - _N),
                      constant_values=_G).reshape(1, _NP)

    hT = _tc_transpose(jnp.pad(x, ((0, _NP - _N), (0, 0))))
    for Wr, br, Wt in [(W_rel0, b_rel0, W_root0), (W_rel1, b_rel1, W_root1)]:
        p = _sc_agg(hT.reshape(-1), edata).reshape(_D, _NP)
        hT = _tc_layer(p, hT, Wr, Wt, br.reshape(_D, 1))

    p = _sc_agg(hT.reshape(-1), edata).reshape(_D, _NP)
    return _layer_pool_head(p, hT, W_rel2, W_root2, b_rel2.reshape(_D, 1),
                            batch2d, lin1_W, lin1_b.reshape(1, _D),
                            lin2_W, lin2_b.reshape(1, _OUT))


# speedup vs baseline: 1.1217x; 1.1217x over previous
import functools

import jax
import jax.numpy as jnp
from jax import lax
from jax.experimental import pallas as pl
from jax.experimental.pallas import tpu as pltpu
from jax.experimental.pallas import tpu_sc as plsc

_N = 10000
_NP = 10240
_D = 128
_E = 320000
_G = 64
_OUT = 64
_ECH = 512
_NCH = _E // _ECH
_CPT = 4
_W = _CPT * _NP
_ERS = 2 * _ECH


def _sc_agg_body(h_hbm, e_hbm, out_hbm, ering, hcol, acc, e0, e1, e2, e3):
    c = lax.axis_index("c")
    s = lax.axis_index("s")
    t = s * 2 + c
    esems = (e0, e1, e2, e3)
    zero16 = jnp.zeros((16,), jnp.float32)
    iota16 = lax.iota(jnp.int32, 16)

    pltpu.sync_copy(h_hbm.at[pl.ds(t * (_W // 2), _W // 2)], hcol)

    @plsc.parallel_loop(0, _W // 16)
    def _z(i):
        acc[pl.ds(i * 16, 16)] = zero16

    def refill(j, slot):
        pltpu.async_copy(e_hbm.at[jnp.minimum(j, _NCH - 1), 0],
                         ering.at[pl.ds(_ERS * slot, _ERS)], esems[slot])

    def wait_refill(slot):
        pltpu.make_async_copy(e_hbm.at[0, 0],
                              ering.at[pl.ds(_ERS * slot, _ERS)],
                              esems[slot]).wait()

    def do_chunk(slot):
        sb = _ERS * slot

        @plsc.parallel_loop(0, _ECH // 16, unroll=2)
        def grp(g):
            gb = g * 16
            pair = plsc.load_gather(ering, [iota16 + (sb + gb)])
            src16 = jnp.bitwise_and(pair, 0xFFFF)
            dst16 = lax.shift_right_logical(pair, 16)
            w16 = plsc.bitcast(
                plsc.load_gather(ering, [iota16 + (sb + _ECH + gb)]),
                jnp.float32)
            for j in range(2):
                joff = jnp.full((16,), j * _NP, jnp.int32)
                pw = plsc.load_gather(hcol, [src16 + joff])
                vlo = plsc.bitcast(lax.shift_left(pw, 16), jnp.float32) * w16
                vhi = plsc.bitcast(
                    jnp.bitwise_and(pw, jnp.int32(-65536)), jnp.float32
                ) * w16
                plsc.addupdate_scatter(acc, [dst16 + joff], vlo)
                plsc.addupdate_scatter(
                    acc, [dst16 + jnp.full((16,), (2 + j) * _NP, jnp.int32)],
                    vhi)

    for slot in range(4):
        refill(slot, slot)

    def rnd(r, carry):
        for slot in range(4):
            wait_refill(slot)
            do_chunk(slot)
            refill(4 * r + slot + 4, slot)
        return carry

    lax.fori_loop(0, _NCH // 4, rnd, 0)
    wait_refill(0)
    do_chunk(0)
    for slot in range(1, 4):
        wait_refill(slot)

    half = _W // 2
    pltpu.sync_copy(acc.at[pl.ds(0, half)],
                    out_hbm.at[pl.ds(2 * t * _NP, half)])
    pltpu.sync_copy(acc.at[pl.ds(half, half)],
                    out_hbm.at[pl.ds((64 + 2 * t) * _NP, half)])


def _sc_agg(hT_flat, edata):
    return pl.kernel(
        _sc_agg_body,
        out_type=jax.ShapeDtypeStruct((_D * _NP,), jnp.float32),
        mesh=plsc.VectorSubcoreMesh(core_axis_name="c", subcore_axis_name="s"),
        scratch_types=[
            pltpu.VMEM((4 * _ERS,), jnp.int32),
            pltpu.VMEM((_W // 2,), jnp.int32),
            pltpu.VMEM((_W,), jnp.float32),
            pltpu.SemaphoreType.DMA,
            pltpu.SemaphoreType.DMA,
            pltpu.SemaphoreType.DMA,
            pltpu.SemaphoreType.DMA,
        ],
        compiler_params=pltpu.CompilerParams(needs_layout_passes=False),
    )(hT_flat, edata)


_TDN = (((0,), (0,)), ((), ()))


def _pack_bf16_pairs(hT):
    a = lax.bitcast_convert_type(hT[:64].astype(jnp.bfloat16),
                                 jnp.uint16).astype(jnp.uint32)
    b = lax.bitcast_convert_type(hT[64:].astype(jnp.bfloat16),
                                 jnp.uint16).astype(jnp.uint32)
    return lax.bitcast_convert_type(a | (b << 16), jnp.int32)


def _tc_transpose_body(x_ref, o_ref, p_ref):
    xT = x_ref[...].T
    o_ref[...] = xT
    p_ref[...] = _pack_bf16_pairs(xT)


def _tc_transpose(x):
    return pl.pallas_call(
        _tc_transpose_body,
        out_shape=[jax.ShapeDtypeStruct((_D, _NP), jnp.float32),
                   jax.ShapeDtypeStruct((_D // 2, _NP), jnp.int32)],
    )(x)


def _tc_layer_body(p_ref, h_ref, wr_ref, wt_ref, b_ref, o_ref, op_ref):
    out = lax.dot_general(wr_ref[...], p_ref[...], _TDN,
                          preferred_element_type=jnp.float32)
    out = out + lax.dot_general(wt_ref[...], h_ref[...], _TDN,
                                preferred_element_type=jnp.float32)
    out = jnp.maximum(out + b_ref[...], 0.0)
    o_ref[...] = out
    op_ref[...] = _pack_bf16_pairs(out)


def _tc_layer(p, h, Wr, Wt, b):
    return pl.pallas_call(
        _tc_layer_body,
        out_shape=[jax.ShapeDtypeStruct((_D, _NP), jnp.float32),
                   jax.ShapeDtypeStruct((_D // 2, _NP), jnp.int32)],
    )(p, h, Wr, Wt, b)


def _layer_pool_head_body(p_ref, h_ref, wr_ref, wt_ref, b_ref, b2d_ref,
                          w1_ref, b1_ref, w2_ref, b2_ref, o_ref):
    hblk = lax.dot_general(wr_ref[...], p_ref[...], _TDN,
                           preferred_element_type=jnp.float32)
    hblk = hblk + lax.dot_general(wt_ref[...], h_ref[...], _TDN,
                                  preferred_element_type=jnp.float32)
    hblk = hblk + b_ref[...]

    bvec = b2d_ref[...]
    oh = (lax.broadcasted_iota(jnp.int32, (_G, _NP), 0) == bvec
          ).astype(jnp.float32)
    sums = lax.dot_general(oh, hblk, (((1,), (1,)), ((), ())),
                           preferred_element_type=jnp.float32)
    cnts = jnp.sum(oh, axis=1, keepdims=True)
    pooled = sums / jnp.maximum(cnts, 1.0)
    r = jnp.dot(pooled, w1_ref[...], preferred_element_type=jnp.float32)
    r = jnp.maximum(r + b1_ref[...], 0.0)
    o_ref[...] = (jnp.dot(r, w2_ref[...], preferred_element_type=jnp.float32)
                  + b2_ref[...])


def _layer_pool_head(p, h, Wr, Wt, b, batch2d, w1, b1, w2, b2):
    return pl.pallas_call(
        _layer_pool_head_body,
        out_shape=jax.ShapeDtypeStruct((_G, _OUT), jnp.float32),
    )(p, h, Wr, Wt, b, batch2d, w1, b1, w2, b2)


def kernel(x, edge_index, edge_weight, batch, W_rel0, b_rel0, W_root0,
           W_rel1, b_rel1, W_root1, W_rel2, b_rel2, W_root2,
           lin1_W, lin1_b, lin2_W, lin2_b):
    wsi = lax.bitcast_convert_type(edge_weight, jnp.int32)
    pair = jnp.bitwise_or(edge_index[0],
                          jnp.left_shift(edge_index[1], 16))
    edata = jnp.concatenate(
        [pair.reshape(_NCH, 1, _ECH),
         wsi.reshape(_NCH, 1, _ECH)], axis=1).reshape(_NCH, 1, _ERS)
    batch2d = jnp.pad(batch, (0, _NP - _N),
                      constant_values=_G).reshape(1, _NP)

    hT, hP = _tc_transpose(jnp.pad(x, ((0, _NP - _N), (0, 0))))
    for Wr, br, Wt in [(W_rel0, b_rel0, W_root0), (W_rel1, b_rel1, W_root1)]:
        p = _sc_agg(hP.reshape(-1), edata).reshape(_D, _NP)
        hT, hP = _tc_layer(p, hT, Wr, Wt, br.reshape(_D, 1))

    p = _sc_agg(hP.reshape(-1), edata).reshape(_D, _NP)
    return _layer_pool_head(p, hT, W_rel2, W_root2, b_rel2.reshape(_D, 1),
                            batch2d, lin1_W, lin1_b.reshape(1, _D),
                            lin2_W, lin2_b.reshape(1, _OUT))


# speedup vs baseline: 1.1591x; 1.0333x over previous
import functools

import jax
import jax.numpy as jnp
from jax import lax
from jax.experimental import pallas as pl
from jax.experimental.pallas import tpu as pltpu
from jax.experimental.pallas import tpu_sc as plsc

_N = 10000
_NP = 10240
_D = 128
_E = 320000
_G = 64
_OUT = 64
_ECH = 640
_NCH = _E // _ECH
_CPT = 4
_W = _CPT * _NP
_ERS = 2 * _ECH


def _sc_agg_body(h_hbm, e_hbm, out_hbm, ering, hcol, acc, e0, e1, e2, e3):
    c = lax.axis_index("c")
    s = lax.axis_index("s")
    t = s * 2 + c
    esems = (e0, e1, e2, e3)
    zero16 = jnp.zeros((16,), jnp.float32)
    iota16 = lax.iota(jnp.int32, 16)

    pltpu.sync_copy(h_hbm.at[pl.ds(t * (_W // 2), _W // 2)], hcol)

    @plsc.parallel_loop(0, _W // 16)
    def _z(i):
        acc[pl.ds(i * 16, 16)] = zero16

    def refill(j, slot):
        pltpu.async_copy(e_hbm.at[jnp.minimum(j, _NCH - 1), 0],
                         ering.at[pl.ds(_ERS * slot, _ERS)], esems[slot])

    def wait_refill(slot):
        pltpu.make_async_copy(e_hbm.at[0, 0],
                              ering.at[pl.ds(_ERS * slot, _ERS)],
                              esems[slot]).wait()

    def do_chunk(slot):
        sb = _ERS * slot

        @plsc.parallel_loop(0, _ECH // 16, unroll=2)
        def grp(g):
            gb = g * 16
            pair = plsc.load_gather(ering, [iota16 + (sb + gb)])
            src16 = jnp.bitwise_and(pair, 0xFFFF)
            dst16 = lax.shift_right_logical(pair, 16)
            w16 = plsc.bitcast(
                plsc.load_gather(ering, [iota16 + (sb + _ECH + gb)]),
                jnp.float32)
            for j in range(2):
                joff = jnp.full((16,), j * _NP, jnp.int32)
                pw = plsc.load_gather(hcol, [src16 + joff])
                vlo = plsc.bitcast(lax.shift_left(pw, 16), jnp.float32) * w16
                vhi = plsc.bitcast(
                    jnp.bitwise_and(pw, jnp.int32(-65536)), jnp.float32
                ) * w16
                plsc.addupdate_scatter(acc, [dst16 + joff], vlo)
                plsc.addupdate_scatter(
                    acc, [dst16 + jnp.full((16,), (2 + j) * _NP, jnp.int32)],
                    vhi)

    for slot in range(4):
        refill(slot, slot)

    def rnd(r, carry):
        for slot in range(4):
            wait_refill(slot)
            do_chunk(slot)
            refill(4 * r + slot + 4, slot)
        return carry

    lax.fori_loop(0, _NCH // 4 - 1, rnd, 0)
    for slot in range(4):
        wait_refill(slot)
        do_chunk(slot)

    half = _W // 2
    pltpu.sync_copy(acc.at[pl.ds(0, half)],
                    out_hbm.at[pl.ds(2 * t * _NP, half)])
    pltpu.sync_copy(acc.at[pl.ds(half, half)],
                    out_hbm.at[pl.ds((64 + 2 * t) * _NP, half)])


def _sc_agg(hT_flat, edata):
    return pl.kernel(
        _sc_agg_body,
        out_type=jax.ShapeDtypeStruct((_D * _NP,), jnp.float32),
        mesh=plsc.VectorSubcoreMesh(core_axis_name="c", subcore_axis_name="s"),
        scratch_types=[
            pltpu.VMEM((4 * _ERS,), jnp.int32),
            pltpu.VMEM((_W // 2,), jnp.int32),
            pltpu.VMEM((_W,), jnp.float32),
            pltpu.SemaphoreType.DMA,
            pltpu.SemaphoreType.DMA,
            pltpu.SemaphoreType.DMA,
            pltpu.SemaphoreType.DMA,
        ],
        compiler_params=pltpu.CompilerParams(needs_layout_passes=False),
    )(hT_flat, edata)


_TDN = (((0,), (0,)), ((), ()))


def _pack_bf16_pairs(hT):
    a = lax.bitcast_convert_type(hT[:64].astype(jnp.bfloat16),
                                 jnp.uint16).astype(jnp.uint32)
    b = lax.bitcast_convert_type(hT[64:].astype(jnp.bfloat16),
                                 jnp.uint16).astype(jnp.uint32)
    return lax.bitcast_convert_type(a | (b << 16), jnp.int32)


def _tc_transpose_body(x_ref, o_ref, p_ref):
    xT = x_ref[...].T
    o_ref[...] = xT
    p_ref[...] = _pack_bf16_pairs(xT)


def _tc_transpose(x):
    return pl.pallas_call(
        _tc_transpose_body,
        out_shape=[jax.ShapeDtypeStruct((_D, _NP), jnp.float32),
                   jax.ShapeDtypeStruct((_D // 2, _NP), jnp.int32)],
    )(x)


def _tc_layer_body(p_ref, h_ref, wr_ref, wt_ref, b_ref, o_ref, op_ref):
    out = lax.dot_general(wr_ref[...], p_ref[...], _TDN,
                          preferred_element_type=jnp.float32)
    out = out + lax.dot_general(wt_ref[...], h_ref[...], _TDN,
                                preferred_element_type=jnp.float32)
    out = jnp.maximum(out + b_ref[...], 0.0)
    o_ref[...] = out
    op_ref[...] = _pack_bf16_pairs(out)


def _tc_layer(p, h, Wr, Wt, b):
    return pl.pallas_call(
        _tc_layer_body,
        out_shape=[jax.ShapeDtypeStruct((_D, _NP), jnp.float32),
                   jax.ShapeDtypeStruct((_D // 2, _NP), jnp.int32)],
    )(p, h, Wr, Wt, b)


def _layer_pool_head_body(p_ref, h_ref, wr_ref, wt_ref, b_ref, b2d_ref,
                          w1_ref, b1_ref, w2_ref, b2_ref, o_ref):
    hblk = lax.dot_general(wr_ref[...], p_ref[...], _TDN,
                           preferred_element_type=jnp.float32)
    hblk = hblk + lax.dot_general(wt_ref[...], h_ref[...], _TDN,
                                  preferred_element_type=jnp.float32)
    hblk = hblk + b_ref[...]

    bvec = b2d_ref[...]
    oh = (lax.broadcasted_iota(jnp.int32, (_G, _NP), 0) == bvec
          ).astype(jnp.float32)
    sums = lax.dot_general(oh, hblk, (((1,), (1,)), ((), ())),
                           preferred_element_type=jnp.float32)
    cnts = jnp.sum(oh, axis=1, keepdims=True)
    pooled = sums / jnp.maximum(cnts, 1.0)
    r = jnp.dot(pooled, w1_ref[...], preferred_element_type=jnp.float32)
    r = jnp.maximum(r + b1_ref[...], 0.0)
    o_ref[...] = (jnp.dot(r, w2_ref[...], preferred_element_type=jnp.float32)
                  + b2_ref[...])


def _layer_pool_head(p, h, Wr, Wt, b, batch2d, w1, b1, w2, b2):
    return pl.pallas_call(
        _layer_pool_head_body,
        out_shape=jax.ShapeDtypeStruct((_G, _OUT), jnp.float32),
    )(p, h, Wr, Wt, b, batch2d, w1, b1, w2, b2)


def kernel(x, edge_index, edge_weight, batch, W_rel0, b_rel0, W_root0,
           W_rel1, b_rel1, W_root1, W_rel2, b_rel2, W_root2,
           lin1_W, lin1_b, lin2_W, lin2_b):
    wsi = lax.bitcast_convert_type(edge_weight, jnp.int32)
    pair = jnp.bitwise_or(edge_index[0],
                          jnp.left_shift(edge_index[1], 16))
    edata = jnp.concatenate(
        [pair.reshape(_NCH, 1, _ECH),
         wsi.reshape(_NCH, 1, _ECH)], axis=1).reshape(_NCH, 1, _ERS)
    batch2d = jnp.pad(batch, (0, _NP - _N),
                      constant_values=_G).reshape(1, _NP)

    hT, hP = _tc_transpose(jnp.pad(x, ((0, _NP - _N), (0, 0))))
    for Wr, br, Wt in [(W_rel0, b_rel0, W_root0), (W_rel1, b_rel1, W_root1)]:
        p = _sc_agg(hP.reshape(-1), edata).reshape(_D, _NP)
        hT, hP = _tc_layer(p, hT, Wr, Wt, br.reshape(_D, 1))

    p = _sc_agg(hP.reshape(-1), edata).reshape(_D, _NP)
    return _layer_pool_head(p, hT, W_rel2, W_root2, b_rel2.reshape(_D, 1),
                            batch2d, lin1_W, lin1_b.reshape(1, _D),
                            lin2_W, lin2_b.reshape(1, _OUT))
